# Initial kernel scaffold; baseline (speedup 1.0000x reference)
import jax
import jax.numpy as jnp
from jax.experimental import pallas as pl


def kernel(x, pos, edge_index, edge_attr, params):
    raise NotImplementedError("write your pallas kernel here")


# speedup vs baseline: 1.8606x; 1.8606x over previous
import functools

import jax
import jax.numpy as jnp
import numpy as np
from jax.experimental import pallas as pl
from jax.experimental.pallas import tpu as pltpu

_N = 10000
_E = 160000
_NODE_DIM = 128
_EDGE_DIM = 16
_HIDDEN = 128
_NUM_LAYERS = 3
_NUM_RADIAL = 16
_NUM_ANGULAR = 8
_CUTOFF = 8.0
_CAP = _E * 32


def _linear_kernel(x_ref, wt_ref, b_ref, o_ref, *, act):
    y = jax.lax.dot_general(
        x_ref[...], wt_ref[...], (((1,), (0,)), ((), ())),
        preferred_element_type=jnp.float32,
        precision=jax.lax.Precision.HIGHEST,
    ) + b_ref[...]
    if act:
        y = y * jax.nn.sigmoid(y)
    o_ref[...] = y


def _linear_pallas(x, w, b, act=False, block_m=1000):
    m, k = x.shape
    o = w.shape[0]
    pad = (-m) % block_m
    if pad:
        x = jnp.concatenate([x, jnp.zeros((pad, k), x.dtype)], axis=0)
    mp = x.shape[0]
    wt = w.T
    b2 = b[None, :]
    out = pl.pallas_call(
        functools.partial(_linear_kernel, act=act),
        grid=(mp // block_m,),
        in_specs=[
            pl.BlockSpec((block_m, k), lambda i: (i, 0)),
            pl.BlockSpec((k, o), lambda i: (0, 0)),
            pl.BlockSpec((1, o), lambda i: (0, 0)),
        ],
        out_specs=pl.BlockSpec((block_m, o), lambda i: (i, 0)),
        out_shape=jax.ShapeDtypeStruct((mp, o), jnp.float32),
    )(x, wt, b2)
    return out[:m] if pad else out


def kernel(x, pos, edge_index, edge_attr, params):
    row = edge_index[0]
    col = edge_index[1]
    h = _linear_pallas(x, params['node_proj_w'], params['node_proj_b'])

    rel = pos[row] - pos[col]
    dist = jnp.sqrt(jnp.sum(rel * rel, axis=-1))
    centers_r = jnp.linspace(0.0, _CUTOFF, _NUM_RADIAL)
    wr = centers_r[1] - centers_r[0] + 1e-12
    radial = jnp.exp(-(dist[:, None] - centers_r[None, :]) ** 2 / wr ** 2)

    order = jnp.argsort(col)
    s_row = row[order]
    counts = jnp.bincount(col, length=_N)
    starts = jnp.concatenate([jnp.zeros((1,), counts.dtype), jnp.cumsum(counts)[:-1]])
    reps = counts[col]
    total = reps.sum()
    edge_id = jnp.repeat(jnp.arange(_E), reps, total_repeat_length=_CAP)
    grp_starts = jnp.concatenate([jnp.zeros((1,), reps.dtype), jnp.cumsum(reps)[:-1]])
    local = jnp.arange(_CAP) - jnp.repeat(grp_starts, reps, total_repeat_length=_CAP)
    in_range = jnp.arange(_CAP) < total
    k_pos = jnp.clip(starts[col[edge_id]] + local, 0, _E - 1)
    k_node = s_row[k_pos]
    i_node = row[edge_id]
    j_node = col[edge_id]
    valid = (k_node != i_node) & in_range
    v1 = pos[i_node] - pos[j_node]
    v2 = pos[k_node] - pos[j_node]
    v1n = v1 / (jnp.sqrt(jnp.sum(v1 * v1, axis=-1, keepdims=True)) + 1e-08)
    v2n = v2 / (jnp.sqrt(jnp.sum(v2 * v2, axis=-1, keepdims=True)) + 1e-08)
    cos_theta = jnp.sum(v1n * v2n, axis=-1)
    centers_a = jnp.linspace(-1.0, 1.0, _NUM_ANGULAR)
    wa = centers_a[1] - centers_a[0] + 1e-12
    ang = jnp.exp(-(cos_theta[:, None] - centers_a[None, :]) ** 2 / wa ** 2)
    ang = ang * valid[:, None].astype(ang.dtype)
    ang_sum = jax.ops.segment_sum(ang, edge_id, num_segments=_E)
    cnt = jax.ops.segment_sum(valid.astype(jnp.float32), edge_id, num_segments=_E)
    angular = ang_sum / jnp.maximum(cnt, 1.0)[:, None]

    edge_feat_raw = jnp.concatenate([radial, angular, edge_attr], axis=-1)
    edge_feat = _linear_pallas(edge_feat_raw, params['edge_proj_w'], params['edge_proj_b'])

    for l in range(_NUM_LAYERS):
        m_in = jnp.concatenate([h[row], h[col], edge_feat], axis=-1)
        m = _linear_pallas(m_in, params['l%d_ew1' % l], params['l%d_eb1' % l], act=True)
        m = _linear_pallas(m, params['l%d_ew2' % l], params['l%d_eb2' % l], act=True)
        agg = jnp.zeros((_N, _HIDDEN), dtype=m.dtype).at[row].add(m)
        hcat = jnp.concatenate([h, agg], axis=-1)
        hmid = _linear_pallas(hcat, params['l%d_nw1' % l], params['l%d_nb1' % l], act=True)
        h = _linear_pallas(hmid, params['l%d_nw2' % l], params['l%d_nb2' % l])
    local_se = h

    g = jnp.mean(h, axis=0, keepdims=True)
    global_se = g @ params['global_w'].T + params['global_b'][None, :]
    return local_se, global_se, pos


# speedup vs baseline: 105.1207x; 56.4996x over previous
import functools

import jax
import jax.numpy as jnp
from jax import lax
from jax.experimental import pallas as pl

_N = 10000
_E = 160000
_HIDDEN = 128
_NUM_LAYERS = 3
_NUM_RADIAL = 16
_NUM_ANGULAR = 8
_CUTOFF = 8.0
_CAP = _E * 32

_K = 64
_BN = 8


def _linear_kernel(x_ref, wt_ref, b_ref, o_ref, *, act):
    y = lax.dot_general(
        x_ref[...], wt_ref[...], (((1,), (0,)), ((), ())),
        preferred_element_type=jnp.float32,
        precision=lax.Precision.HIGHEST,
    ) + b_ref[...]
    if act:
        y = y * jax.nn.sigmoid(y)
    o_ref[...] = y


def _linear_pallas(x, w, b, act=False, block_m=1000):
    m, k = x.shape
    o = w.shape[0]
    pad = (-m) % block_m
    if pad:
        x = jnp.concatenate([x, jnp.zeros((pad, k), x.dtype)], axis=0)
    mp = x.shape[0]
    out = pl.pallas_call(
        functools.partial(_linear_kernel, act=act),
        grid=(mp // block_m,),
        in_specs=[
            pl.BlockSpec((block_m, k), lambda i: (i, 0)),
            pl.BlockSpec((k, o), lambda i: (0, 0)),
            pl.BlockSpec((1, o), lambda i: (0, 0)),
        ],
        out_specs=pl.BlockSpec((block_m, o), lambda i: (i, 0)),
        out_shape=jax.ShapeDtypeStruct((mp, o), jnp.float32),
    )(x, w.T, b[None, :])
    return out[:m] if pad else out


def _angular_kernel(ux_ref, uy_ref, uz_ref, s_ref, t_ref, asum_ref, cnt_ref):
    centers = [(-1.0 + 2.0 * i / (_NUM_ANGULAR - 1)) for i in range(_NUM_ANGULAR)]
    wa = 2.0 / (_NUM_ANGULAR - 1) + 1e-12
    neg_inv_wa2 = -1.0 / (wa * wa)
    ux = ux_ref[...]
    uy = uy_ref[...]
    uz = uz_ref[...]
    s = s_ref[...]
    t = t_ref[...]
    gram = (ux[:, :, None] * ux[:, None, :]
            + uy[:, :, None] * uy[:, None, :]
            + uz[:, :, None] * uz[:, None, :])
    l_iota = lax.broadcasted_iota(jnp.int32, (_BN, _K, _K), 2)
    msk = (l_iota < t[:, :, None]) & (s[:, :, None] != s[:, None, :])
    cosm = jnp.where(msk, gram, jnp.float32(3.0))
    accs = []
    for c in centers:
        d = cosm - jnp.float32(c)
        accs.append(jnp.sum(jnp.exp(d * d * jnp.float32(neg_inv_wa2)), axis=2))
    asum_ref[...] = jnp.stack(accs, axis=-1)
    cnt_ref[...] = jnp.sum(msk.astype(jnp.float32), axis=2)


def _angular_pallas(ux, uy, uz, s, t):
    return pl.pallas_call(
        _angular_kernel,
        grid=(_N // _BN,),
        in_specs=[pl.BlockSpec((_BN, _K), lambda i: (i, 0))] * 5,
        out_specs=[
            pl.BlockSpec((_BN, _K, _NUM_ANGULAR), lambda i: (i, 0, 0)),
            pl.BlockSpec((_BN, _K), lambda i: (i, 0)),
        ],
        out_shape=[
            jax.ShapeDtypeStruct((_N, _K, _NUM_ANGULAR), jnp.float32),
            jax.ShapeDtypeStruct((_N, _K), jnp.float32),
        ],
    )(ux, uy, uz, s, t)


def kernel(x, pos, edge_index, edge_attr, params):
    row = edge_index[0]
    col = edge_index[1]

    order = jnp.argsort(col)
    scol = col[order]
    srow = row[order]
    counts = jnp.bincount(col, length=_N)
    starts = jnp.concatenate([jnp.zeros((1,), counts.dtype), jnp.cumsum(counts)[:-1]])
    reps = counts[col]
    grp_starts = jnp.concatenate([jnp.zeros((1,), reps.dtype), jnp.cumsum(reps)[:-1]])
    t_orig = jnp.minimum(reps, jnp.maximum(0, _CAP - grp_starts)).astype(jnp.int32)
    t_s = t_orig[order]

    rel = pos[srow] - pos[scol]
    dist = jnp.sqrt(jnp.sum(rel * rel, axis=-1))
    vn = rel / (dist[:, None] + 1e-08)

    rank = (jnp.arange(_E, dtype=jnp.int32) - starts[scol].astype(jnp.int32))
    slot = jnp.where(rank < _K, scol.astype(jnp.int32) * _K + rank, _N * _K)
    zslots = jnp.zeros((_N * _K,), jnp.float32)
    ux = zslots.at[slot].set(vn[:, 0], mode='drop').reshape(_N, _K)
    uy = zslots.at[slot].set(vn[:, 1], mode='drop').reshape(_N, _K)
    uz = zslots.at[slot].set(vn[:, 2], mode='drop').reshape(_N, _K)
    s_pad = jnp.full((_N * _K,), -1, jnp.int32).at[slot].set(
        srow.astype(jnp.int32), mode='drop').reshape(_N, _K)
    t_pad = jnp.zeros((_N * _K,), jnp.int32).at[slot].set(
        t_s, mode='drop').reshape(_N, _K)

    asum_n, cnt_n = _angular_pallas(ux, uy, uz, s_pad, t_pad)
    gslot = jnp.minimum(slot, _N * _K - 1)
    ang_sum = asum_n.reshape(_N * _K, _NUM_ANGULAR)[gslot]
    cnt = cnt_n.reshape(_N * _K)[gslot]
    angular = ang_sum / jnp.maximum(cnt, 1.0)[:, None]

    centers_r = jnp.linspace(0.0, _CUTOFF, _NUM_RADIAL)
    wr = centers_r[1] - centers_r[0] + 1e-12
    radial = jnp.exp(-(dist[:, None] - centers_r[None, :]) ** 2 / wr ** 2)

    h = _linear_pallas(x, params['node_proj_w'], params['node_proj_b'])
    edge_attr_s = edge_attr[order]
    edge_feat_raw = jnp.concatenate([radial, angular, edge_attr_s], axis=-1)
    edge_feat = _linear_pallas(edge_feat_raw, params['edge_proj_w'], params['edge_proj_b'])

    for l in range(_NUM_LAYERS):
        m_in = jnp.concatenate([h[srow], h[scol], edge_feat], axis=-1)
        m = _linear_pallas(m_in, params['l%d_ew1' % l], params['l%d_eb1' % l], act=True)
        m = _linear_pallas(m, params['l%d_ew2' % l], params['l%d_eb2' % l], act=True)
        agg = jnp.zeros((_N, _HIDDEN), dtype=m.dtype).at[srow].add(m)
        hcat = jnp.concatenate([h, agg], axis=-1)
        hmid = _linear_pallas(hcat, params['l%d_nw1' % l], params['l%d_nb1' % l], act=True)
        h = _linear_pallas(hmid, params['l%d_nw2' % l], params['l%d_nb2' % l])
    local_se = h

    g = jnp.mean(h, axis=0, keepdims=True)
    global_se = g @ params['global_w'].T + params['global_b'][None, :]
    return local_se, global_se, pos
